# baseline (speedup 1.0000x reference)
import functools
import math

import jax
import jax.numpy as jnp
from jax import lax
from jax.experimental import pallas as pl
from jax.experimental.pallas import tpu as pltpu
from jax.experimental.pallas import tpu_sc as plsc

N = 10000
E = 160000
C = 4
D = 128
NG = 5
NP = 10240
BN = 512
NC = 2
NS = 16
CH = 128
EPC = E // NC
SUB = 5120

_f32 = jnp.float32


def _deg_counts(dst_all):
  mesh = plsc.VectorSubcoreMesh(core_axis_name="c", subcore_axis_name="s")

  @functools.partial(
      pl.kernel,
      mesh=mesh,
      out_type=jax.ShapeDtypeStruct((NC * NG * NP,), _f32),
      scratch_types=[
          pltpu.VMEM((SUB,), jnp.int32),
          pltpu.VMEM((CH,), jnp.int32),
          pltpu.VMEM((CH,), _f32),
          pltpu.VMEM((1600,), _f32),
          pltpu.VMEM_SHARED((NG * NP,), _f32),
      ],
  )
  def body(dst_hbm, out_hbm, dbuf, gidx, ones, zb, deg):
    c = lax.axis_index("c")
    s = lax.axis_index("s")
    for q in range(CH // 16):
      ones[pl.ds(q * 16, 16)] = jnp.full((16,), 1.0, _f32)

    def zrow(j, _):
      zb[pl.ds(j * 16, 16)] = jnp.zeros((16,), _f32)
      return 0

    lax.fori_loop(0, 100, zrow, 0)
    for t in range(2):
      pltpu.sync_copy(zb, deg.at[pl.ds(s * 3200 + t * 1600, 1600)])
    plsc.subcore_barrier()

    base = c * EPC + s * SUB
    ne = jnp.where(s < NS - 1, SUB, EPC - (NS - 1) * SUB)
    nk = ne // CH
    for g in range(NG):
      pltpu.sync_copy(
          dst_hbm.at[pl.ds(pl.multiple_of(g * E + base, 8), SUB)], dbuf)

      def chunk(k, _):
        for q in range(CH // 16):
          gidx[pl.ds(q * 16, 16)] = dbuf[pl.ds(k * CH + q * 16, 16)] + g * NP
        pltpu.sync_copy(ones, deg.at[gidx], add=True)
        return 0

      lax.fori_loop(0, nk, chunk, 0)
    plsc.subcore_barrier()
    for t in range(2):
      o = s * 3200 + t * 1600
      pltpu.sync_copy(deg.at[pl.ds(o, 1600)], zb)
      pltpu.sync_copy(zb, out_hbm.at[pl.ds(c * (NG * NP) + o, 1600)])

  return body(dst_all)


def _edge_aggregate(tmp_all, src_all, dst_all):
  mesh = plsc.VectorSubcoreMesh(core_axis_name="c", subcore_axis_name="s")

  @functools.partial(
      pl.kernel,
      mesh=mesh,
      out_type=jax.ShapeDtypeStruct((NC, NG, NP, D), _f32),
      scratch_types=[
          pltpu.VMEM((SUB,), jnp.int32),
          pltpu.VMEM((SUB,), jnp.int32),
          pltpu.VMEM((CH,), jnp.int32),
          pltpu.VMEM((CH,), jnp.int32),
          pltpu.VMEM((CH, D), _f32),
          pltpu.VMEM((CH, D), _f32),
          pltpu.VMEM_SHARED((NP // NS, D), _f32),
          pltpu.VMEM_SHARED((NP, D), _f32),
          pltpu.SemaphoreType.DMA,
          pltpu.SemaphoreType.DMA,
      ],
  )
  def body(tmp_hbm, src_hbm, dst_hbm, out_hbm, sbuf, dbuf, didx0, didx1,
           rows0, rows1, zshared, agg, sem0, sem1):
    c = lax.axis_index("c")
    s = lax.axis_index("s")
    rpw = NP // NS

    def zrow(r, _):
      for q in range(D // 16):
        rows0[r, pl.ds(q * 16, 16)] = jnp.zeros((16,), _f32)
      return 0

    lax.fori_loop(0, rpw // NS, zrow, 0)
    pltpu.sync_copy(rows0.at[pl.ds(0, rpw // NS)],
                    zshared.at[pl.ds(s * (rpw // NS), rpw // NS)])
    plsc.subcore_barrier()

    base = c * EPC + s * SUB
    ne = jnp.where(s < NS - 1, SUB, EPC - (NS - 1) * SUB)
    nk = ne // CH

    for g in range(NG):
      goff = pl.multiple_of(g * E + base, 8)
      pltpu.sync_copy(src_hbm.at[pl.ds(goff, SUB)], sbuf)
      pltpu.sync_copy(dst_hbm.at[pl.ds(goff, SUB)], dbuf)
      pltpu.sync_copy(zshared, agg.at[pl.ds(s * rpw, rpw)])
      plsc.subcore_barrier()

      def gather(k, rows, sem):
        pltpu.async_copy(tmp_hbm.at[g].at[sbuf.at[pl.ds(k * CH, CH)]],
                         rows, sem)

      def gwait(rows, sem):
        pltpu.make_async_copy(tmp_hbm.at[g].at[sbuf.at[pl.ds(0, CH)]],
                              rows, sem).wait()

      def dcopy(k, didx):
        for q in range(CH // 16):
          didx[pl.ds(q * 16, 16)] = dbuf[pl.ds(k * CH + q * 16, 16)]

      gather(0, rows0, sem0)

      def pair(j, _):
        a = 2 * j

        @pl.when(a + 1 < nk)
        def _():
          gather(a + 1, rows1, sem1)

        dcopy(a, didx0)
        gwait(rows0, sem0)
        pltpu.sync_copy(rows0, agg.at[didx0], add=True)

        @pl.when(a + 2 < nk)
        def _():
          gather(a + 2, rows0, sem0)

        @pl.when(a + 1 < nk)
        def _():
          dcopy(a + 1, didx1)
          gwait(rows1, sem1)
          pltpu.sync_copy(rows1, agg.at[didx1], add=True)

        return 0

      lax.fori_loop(0, (nk + 1) // 2, pair, 0)
      plsc.subcore_barrier()
      pltpu.sync_copy(agg.at[pl.ds(s * rpw, rpw)],
                      out_hbm.at[c, g, pl.ds(s * rpw, rpw)])
      plsc.subcore_barrier()

  return body(tmp_all, src_all, dst_all)


def _tc_pre(degp, gfe0, ofe0, W0):
  grid = (NP // BN,)

  def body(deg_ref, gfe_ref, ofe_ref, w_ref, tmp_ref, dinv_ref):
    deg = deg_ref[...]
    dinv = lax.rsqrt(deg[0] + deg[1] + 1.0)
    w = w_ref[...]
    for g in range(NG):
      x = gfe_ref[g] if g < C else ofe_ref[...]
      h = jnp.dot(x, w, preferred_element_type=_f32)
      tmp_ref[g, :, :] = h * dinv[g][:, None]
    dinv_ref[...] = dinv

  return pl.pallas_call(
      body,
      grid=grid,
      in_specs=[
          pl.BlockSpec((NC, NG, BN), lambda i: (0, 0, i)),
          pl.BlockSpec((C, BN, D), lambda i: (0, i, 0)),
          pl.BlockSpec((BN, D), lambda i: (i, 0)),
          pl.BlockSpec((D, D), lambda i: (0, 0)),
      ],
      out_specs=[
          pl.BlockSpec((NG, BN, D), lambda i: (0, i, 0)),
          pl.BlockSpec((NG, BN), lambda i: (0, i)),
      ],
      out_shape=[
          jax.ShapeDtypeStruct((NG, NP, D), _f32),
          jax.ShapeDtypeStruct((NG, NP), _f32),
      ],
  )(degp, gfe0, ofe0, W0)


def _attention_stage(tmp, agg, dinv, label_emb, Wq, bq, b, Wk, bk, Wv, bv,
                     W_next=None):
  final = W_next is None
  grid = (NP // BN,)
  inv_sqrt_d = 1.0 / math.sqrt(D)

  def body(tmp_ref, agg_ref, dinv_ref, lemb_ref, wq_ref, bq_ref, b_ref,
           wk_ref, bk_ref, wv_ref, bv_ref, *rest):
    if final:
      gfe_ref, ofe_ref = rest
    else:
      (wn_ref, out_ref) = rest
    lq = jnp.dot(lemb_ref[...], wq_ref[...],
                 preferred_element_type=_f32) + bq_ref[...]
    dinv = dinv_ref[...]
    bias = b_ref[...]
    conv = []
    for g in range(NG):
      cg = dinv[g][:, None] * (agg_ref[0, g] + agg_ref[1, g] + tmp_ref[g])
      conv.append(cg + bias)
    wk = wk_ref[...]
    wv = wv_ref[...]
    bk = bk_ref[...]
    bv = bv_ref[...]
    scores = []
    vs = []
    for g in range(C):
      kg = jnp.dot(conv[g], wk, preferred_element_type=_f32) + bk
      vs.append(jnp.dot(conv[g], wv, preferred_element_type=_f32) + bv)
      scores.append([
          jnp.sum(kg * lq[a][None, :], axis=1) * inv_sqrt_d for a in range(C)
      ])
    outs = []
    for a in range(C):
      m = scores[0][a]
      for g in range(1, C):
        m = jnp.maximum(m, scores[g][a])
      es = [jnp.exp(scores[g][a] - m) for g in range(C)]
      z = es[0] + es[1] + es[2] + es[3]
      o = (es[0] / z)[:, None] * vs[0]
      for g in range(1, C):
        o = o + (es[g] / z)[:, None] * vs[g]
      outs.append(o)
    if final:
      for a in range(C):
        gfe_ref[a, :, :] = outs[a]
      ofe_ref[...] = conv[C]
    else:
      wn = wn_ref[...]
      for g in range(NG):
        x2 = jnp.maximum(outs[g] if g < C else conv[C], 0.0)
        h2 = jnp.dot(x2, wn, preferred_element_type=_f32)
        out_ref[g, :, :] = h2 * dinv[g][:, None]

  full = lambda shape: pl.BlockSpec(shape, lambda i: tuple(0 for _ in shape))
  in_specs = [
      pl.BlockSpec((NG, BN, D), lambda i: (0, i, 0)),
      pl.BlockSpec((NC, NG, BN, D), lambda i: (0, 0, i, 0)),
      pl.BlockSpec((NG, BN), lambda i: (0, i)),
      full((C, D)), full((D, D)), full((D,)), full((D,)),
      full((D, D)), full((D,)), full((D, D)), full((D,)),
  ]
  args = [tmp, agg, dinv, label_emb, Wq, bq, b, Wk, bk, Wv, bv]
  if final:
    out_specs = [
        pl.BlockSpec((C, BN, D), lambda i: (0, i, 0)),
        pl.BlockSpec((BN, D), lambda i: (i, 0)),
    ]
    out_shape = [
        jax.ShapeDtypeStruct((C, NP, D), _f32),
        jax.ShapeDtypeStruct((NP, D), _f32),
    ]
  else:
    in_specs.append(full((D, D)))
    args.append(W_next)
    out_specs = [pl.BlockSpec((NG, BN, D), lambda i: (0, i, 0))]
    out_shape = [jax.ShapeDtypeStruct((NG, NP, D), _f32)]
  return pl.pallas_call(
      body, grid=grid, in_specs=in_specs, out_specs=out_specs,
      out_shape=out_shape)(*args)


def kernel(graph_feat_emb, ori_feat_emb, label_emb, edge_index_label,
           edge_index_ori, W0, b0, W1, b1, Wq, bq, Wk, bk, Wv, bv):
  gfe0 = jnp.pad(graph_feat_emb, ((0, 0), (0, NP - N), (0, 0)))
  ofe0 = jnp.pad(ori_feat_emb, ((0, NP - N), (0, 0)))
  eil = edge_index_label.astype(jnp.int32)
  eio = edge_index_ori.astype(jnp.int32)
  src_all = jnp.pad(
      jnp.concatenate([eil[:, 0, :], eio[0:1]], axis=0).reshape(-1), (0, 2048))
  dst_all = jnp.pad(
      jnp.concatenate([eil[:, 1, :], eio[1:2]], axis=0).reshape(-1), (0, 2048))

  degp = _deg_counts(dst_all).reshape(NC, NG, NP)
  tmp1, dinv = _tc_pre(degp, gfe0, ofe0, W0)
  agg1 = _edge_aggregate(tmp1, src_all, dst_all)
  (tmp2,) = _attention_stage(tmp1, agg1, dinv, label_emb, Wq, bq, b0,
                             Wk, bk, Wv, bv, W_next=W1)
  agg2 = _edge_aggregate(tmp2, src_all, dst_all)
  gfe_f, ofe_f = _attention_stage(tmp2, agg2, dinv, label_emb, Wq, bq, b1,
                                  Wk, bk, Wv, bv)
  return gfe_f[:, :N, :], ofe_f[:N, :]


# speedup vs baseline: 1.0005x; 1.0005x over previous
import functools
import math

import jax
import jax.numpy as jnp
from jax import lax
from jax.experimental import pallas as pl
from jax.experimental.pallas import tpu as pltpu
from jax.experimental.pallas import tpu_sc as plsc

N = 10000
E = 160000
C = 4
D = 128
NG = 5
NP = 10240
BN = 512
NC = 2
NS = 16
CH = 128
EPC = E // NC
SUB = 5120

_f32 = jnp.float32


def _deg_counts(dst_all):
  mesh = plsc.VectorSubcoreMesh(core_axis_name="c", subcore_axis_name="s")

  @functools.partial(
      pl.kernel,
      mesh=mesh,
      out_type=jax.ShapeDtypeStruct((NC * NG * NP,), _f32),
      scratch_types=[
          pltpu.VMEM((SUB,), jnp.int32),
          pltpu.VMEM((CH,), jnp.int32),
          pltpu.VMEM((CH,), _f32),
          pltpu.VMEM((1600,), _f32),
          pltpu.VMEM_SHARED((NG * NP,), _f32),
      ],
  )
  def body(dst_hbm, out_hbm, dbuf, gidx, ones, zb, deg):
    c = lax.axis_index("c")
    s = lax.axis_index("s")
    for q in range(CH // 16):
      ones[pl.ds(q * 16, 16)] = jnp.full((16,), 1.0, _f32)

    def zrow(j, _):
      zb[pl.ds(j * 16, 16)] = jnp.zeros((16,), _f32)
      return 0

    lax.fori_loop(0, 100, zrow, 0)
    for t in range(2):
      pltpu.sync_copy(zb, deg.at[pl.ds(s * 3200 + t * 1600, 1600)])
    plsc.subcore_barrier()

    base = c * EPC + s * SUB
    ne = jnp.where(s < NS - 1, SUB, EPC - (NS - 1) * SUB)
    nk = ne // CH
    for g in range(NG):
      pltpu.sync_copy(
          dst_hbm.at[pl.ds(pl.multiple_of(g * E + base, 8), SUB)], dbuf)

      def chunk(k, _):
        for q in range(CH // 16):
          gidx[pl.ds(q * 16, 16)] = dbuf[pl.ds(k * CH + q * 16, 16)] + g * NP
        pltpu.sync_copy(ones, deg.at[gidx], add=True)
        return 0

      lax.fori_loop(0, nk, chunk, 0)
    plsc.subcore_barrier()
    for t in range(2):
      o = s * 3200 + t * 1600
      pltpu.sync_copy(deg.at[pl.ds(o, 1600)], zb)
      pltpu.sync_copy(zb, out_hbm.at[pl.ds(c * (NG * NP) + o, 1600)])

  return body(dst_all)


def _edge_aggregate(tmp_all, src_all, dst_all):
  mesh = plsc.VectorSubcoreMesh(core_axis_name="c", subcore_axis_name="s")

  @functools.partial(
      pl.kernel,
      mesh=mesh,
      out_type=jax.ShapeDtypeStruct((NC, NG, NP, D), _f32),
      scratch_types=[
          pltpu.VMEM((SUB,), jnp.int32),
          pltpu.VMEM((SUB,), jnp.int32),
          pltpu.VMEM((CH,), jnp.int32),
          pltpu.VMEM((CH,), jnp.int32),
          pltpu.VMEM((CH, D), _f32),
          pltpu.VMEM((CH, D), _f32),
          pltpu.VMEM_SHARED((NP // NS, D), _f32),
          pltpu.VMEM_SHARED((NP, D), _f32),
          pltpu.SemaphoreType.DMA,
          pltpu.SemaphoreType.DMA,
      ],
  )
  def body(tmp_hbm, src_hbm, dst_hbm, out_hbm, sbuf, dbuf, didx0, didx1,
           rows0, rows1, zshared, agg, sem0, sem1):
    c = lax.axis_index("c")
    s = lax.axis_index("s")
    rpw = NP // NS

    def zrow(r, _):
      for q in range(D // 16):
        rows0[r, pl.ds(q * 16, 16)] = jnp.zeros((16,), _f32)
      return 0

    lax.fori_loop(0, rpw // NS, zrow, 0)
    pltpu.sync_copy(rows0.at[pl.ds(0, rpw // NS)],
                    zshared.at[pl.ds(s * (rpw // NS), rpw // NS)])
    plsc.subcore_barrier()

    base = c * EPC + s * SUB
    ne = jnp.where(s < NS - 1, SUB, EPC - (NS - 1) * SUB)
    nk = ne // CH

    for g in range(NG):
      goff = pl.multiple_of(g * E + base, 8)
      pltpu.sync_copy(src_hbm.at[pl.ds(goff, SUB)], sbuf)
      pltpu.sync_copy(dst_hbm.at[pl.ds(goff, SUB)], dbuf)
      pltpu.sync_copy(zshared, agg.at[pl.ds(s * rpw, rpw)])
      plsc.subcore_barrier()

      def gather(k, rows, sem):
        pltpu.async_copy(tmp_hbm.at[g].at[sbuf.at[pl.ds(k * CH, CH)]],
                         rows, sem)

      def gwait(rows, sem):
        pltpu.make_async_copy(tmp_hbm.at[g].at[sbuf.at[pl.ds(0, CH)]],
                              rows, sem).wait()

      def dcopy(k, didx):
        for q in range(CH // 16):
          didx[pl.ds(q * 16, 16)] = dbuf[pl.ds(k * CH + q * 16, 16)]

      gather(0, rows0, sem0)

      def pair(j, _):
        a = 2 * j

        @pl.when(a + 1 < nk)
        def _():
          gather(a + 1, rows1, sem1)

        dcopy(a, didx0)
        gwait(rows0, sem0)
        pltpu.sync_copy(rows0, agg.at[didx0], add=True)

        @pl.when(a + 2 < nk)
        def _():
          gather(a + 2, rows0, sem0)

        @pl.when(a + 1 < nk)
        def _():
          dcopy(a + 1, didx1)
          gwait(rows1, sem1)
          pltpu.sync_copy(rows1, agg.at[didx1], add=True)

        return 0

      lax.fori_loop(0, (nk + 1) // 2, pair, 0)
      plsc.subcore_barrier()
      for q in range(rpw // CH):
        r0 = s * rpw + q * CH
        pltpu.sync_copy(agg.at[pl.ds(r0, CH)], rows0)
        pltpu.sync_copy(rows0, out_hbm.at[c, g, pl.ds(r0, CH)])
      plsc.subcore_barrier()

  return body(tmp_all, src_all, dst_all)


def _tc_pre(degp, gfe0, ofe0, W0):
  grid = (NP // BN,)

  def body(deg_ref, gfe_ref, ofe_ref, w_ref, tmp_ref, dinv_ref):
    deg = deg_ref[...]
    dinv = lax.rsqrt(deg[0] + deg[1] + 1.0)
    w = w_ref[...]
    for g in range(NG):
      x = gfe_ref[g] if g < C else ofe_ref[...]
      h = jnp.dot(x, w, preferred_element_type=_f32)
      tmp_ref[g, :, :] = h * dinv[g][:, None]
    dinv_ref[...] = dinv

  return pl.pallas_call(
      body,
      grid=grid,
      in_specs=[
          pl.BlockSpec((NC, NG, BN), lambda i: (0, 0, i)),
          pl.BlockSpec((C, BN, D), lambda i: (0, i, 0)),
          pl.BlockSpec((BN, D), lambda i: (i, 0)),
          pl.BlockSpec((D, D), lambda i: (0, 0)),
      ],
      out_specs=[
          pl.BlockSpec((NG, BN, D), lambda i: (0, i, 0)),
          pl.BlockSpec((NG, BN), lambda i: (0, i)),
      ],
      out_shape=[
          jax.ShapeDtypeStruct((NG, NP, D), _f32),
          jax.ShapeDtypeStruct((NG, NP), _f32),
      ],
  )(degp, gfe0, ofe0, W0)


def _attention_stage(tmp, agg, dinv, label_emb, Wq, bq, b, Wk, bk, Wv, bv,
                     W_next=None):
  final = W_next is None
  grid = (NP // BN,)
  inv_sqrt_d = 1.0 / math.sqrt(D)

  def body(tmp_ref, agg_ref, dinv_ref, lemb_ref, wq_ref, bq_ref, b_ref,
           wk_ref, bk_ref, wv_ref, bv_ref, *rest):
    if final:
      gfe_ref, ofe_ref = rest
    else:
      (wn_ref, out_ref) = rest
    lq = jnp.dot(lemb_ref[...], wq_ref[...],
                 preferred_element_type=_f32) + bq_ref[...]
    dinv = dinv_ref[...]
    bias = b_ref[...]
    conv = []
    for g in range(NG):
      cg = dinv[g][:, None] * (agg_ref[0, g] + agg_ref[1, g] + tmp_ref[g])
      conv.append(cg + bias)
    wk = wk_ref[...]
    wv = wv_ref[...]
    bk = bk_ref[...]
    bv = bv_ref[...]
    scores = []
    vs = []
    for g in range(C):
      kg = jnp.dot(conv[g], wk, preferred_element_type=_f32) + bk
      vs.append(jnp.dot(conv[g], wv, preferred_element_type=_f32) + bv)
      scores.append([
          jnp.sum(kg * lq[a][None, :], axis=1) * inv_sqrt_d for a in range(C)
      ])
    outs = []
    for a in range(C):
      m = scores[0][a]
      for g in range(1, C):
        m = jnp.maximum(m, scores[g][a])
      es = [jnp.exp(scores[g][a] - m) for g in range(C)]
      z = es[0] + es[1] + es[2] + es[3]
      o = (es[0] / z)[:, None] * vs[0]
      for g in range(1, C):
        o = o + (es[g] / z)[:, None] * vs[g]
      outs.append(o)
    if final:
      for a in range(C):
        gfe_ref[a, :, :] = outs[a]
      ofe_ref[...] = conv[C]
    else:
      wn = wn_ref[...]
      for g in range(NG):
        x2 = jnp.maximum(outs[g] if g < C else conv[C], 0.0)
        h2 = jnp.dot(x2, wn, preferred_element_type=_f32)
        out_ref[g, :, :] = h2 * dinv[g][:, None]

  full = lambda shape: pl.BlockSpec(shape, lambda i: tuple(0 for _ in shape))
  in_specs = [
      pl.BlockSpec((NG, BN, D), lambda i: (0, i, 0)),
      pl.BlockSpec((NC, NG, BN, D), lambda i: (0, 0, i, 0)),
      pl.BlockSpec((NG, BN), lambda i: (0, i)),
      full((C, D)), full((D, D)), full((D,)), full((D,)),
      full((D, D)), full((D,)), full((D, D)), full((D,)),
  ]
  args = [tmp, agg, dinv, label_emb, Wq, bq, b, Wk, bk, Wv, bv]
  if final:
    out_specs = [
        pl.BlockSpec((C, BN, D), lambda i: (0, i, 0)),
        pl.BlockSpec((BN, D), lambda i: (i, 0)),
    ]
    out_shape = [
        jax.ShapeDtypeStruct((C, NP, D), _f32),
        jax.ShapeDtypeStruct((NP, D), _f32),
    ]
  else:
    in_specs.append(full((D, D)))
    args.append(W_next)
    out_specs = [pl.BlockSpec((NG, BN, D), lambda i: (0, i, 0))]
    out_shape = [jax.ShapeDtypeStruct((NG, NP, D), _f32)]
  return pl.pallas_call(
      body, grid=grid, in_specs=in_specs, out_specs=out_specs,
      out_shape=out_shape)(*args)


def kernel(graph_feat_emb, ori_feat_emb, label_emb, edge_index_label,
           edge_index_ori, W0, b0, W1, b1, Wq, bq, Wk, bk, Wv, bv):
  gfe0 = jnp.pad(graph_feat_emb, ((0, 0), (0, NP - N), (0, 0)))
  ofe0 = jnp.pad(ori_feat_emb, ((0, NP - N), (0, 0)))
  eil = edge_index_label.astype(jnp.int32)
  eio = edge_index_ori.astype(jnp.int32)
  src_all = jnp.pad(
      jnp.concatenate([eil[:, 0, :], eio[0:1]], axis=0).reshape(-1), (0, 2048))
  dst_all = jnp.pad(
      jnp.concatenate([eil[:, 1, :], eio[1:2]], axis=0).reshape(-1), (0, 2048))

  degp = _deg_counts(dst_all).reshape(NC, NG, NP)
  tmp1, dinv = _tc_pre(degp, gfe0, ofe0, W0)
  agg1 = _edge_aggregate(tmp1, src_all, dst_all)
  (tmp2,) = _attention_stage(tmp1, agg1, dinv, label_emb, Wq, bq, b0,
                             Wk, bk, Wv, bv, W_next=W1)
  agg2 = _edge_aggregate(tmp2, src_all, dst_all)
  gfe_f, ofe_f = _attention_stage(tmp2, agg2, dinv, label_emb, Wq, bq, b1,
                                  Wk, bk, Wv, bv)
  return gfe_f[:, :N, :], ofe_f[:N, :]


# speedup vs baseline: 4.3079x; 4.3058x over previous
import functools
import math

import jax
import jax.numpy as jnp
from jax import lax
from jax.experimental import pallas as pl
from jax.experimental.pallas import tpu as pltpu
from jax.experimental.pallas import tpu_sc as plsc

N = 10000
E = 160000
C = 4
D = 128
NG = 5
NP = 10240
BN = 512
NC = 2
NS = 16
CH = 128
EPC = E // NC
SUB = 5120

_f32 = jnp.float32


def _deg_counts(dst_all):
  mesh = plsc.VectorSubcoreMesh(core_axis_name="c", subcore_axis_name="s")

  @functools.partial(
      pl.kernel,
      mesh=mesh,
      out_type=jax.ShapeDtypeStruct((NC * NG * NP,), _f32),
      scratch_types=[
          pltpu.VMEM((SUB,), jnp.int32),
          pltpu.VMEM((CH,), jnp.int32),
          pltpu.VMEM((CH,), _f32),
          pltpu.VMEM((1600,), _f32),
          pltpu.VMEM_SHARED((NG * NP,), _f32),
      ],
  )
  def body(dst_hbm, out_hbm, dbuf, gidx, ones, zb, deg):
    c = lax.axis_index("c")
    s = lax.axis_index("s")
    for q in range(CH // 16):
      ones[pl.ds(q * 16, 16)] = jnp.full((16,), 1.0, _f32)

    def zrow(j, _):
      zb[pl.ds(j * 16, 16)] = jnp.zeros((16,), _f32)
      return 0

    lax.fori_loop(0, 100, zrow, 0)
    for t in range(2):
      pltpu.sync_copy(zb, deg.at[pl.ds(s * 3200 + t * 1600, 1600)])
    plsc.subcore_barrier()

    base = c * EPC + s * SUB
    ne = jnp.where(s < NS - 1, SUB, EPC - (NS - 1) * SUB)
    nk = ne // CH
    for g in range(NG):
      pltpu.sync_copy(
          dst_hbm.at[pl.ds(pl.multiple_of(g * E + base, 8), SUB)], dbuf)

      def chunk(k, _):
        for q in range(CH // 16):
          gidx[pl.ds(q * 16, 16)] = dbuf[pl.ds(k * CH + q * 16, 16)] + g * NP
        pltpu.sync_copy(ones, deg.at[gidx], add=True)
        return 0

      lax.fori_loop(0, nk, chunk, 0)
    plsc.subcore_barrier()
    for t in range(2):
      o = s * 3200 + t * 1600
      pltpu.sync_copy(deg.at[pl.ds(o, 1600)], zb)
      pltpu.sync_copy(zb, out_hbm.at[pl.ds(c * (NG * NP) + o, 1600)])

  return body(dst_all)


def _edge_aggregate(tmp_all, src_all, dst_all):
  mesh = plsc.VectorSubcoreMesh(core_axis_name="c", subcore_axis_name="s")

  @functools.partial(
      pl.kernel,
      mesh=mesh,
      out_type=jax.ShapeDtypeStruct((NC, NG, NP, D), _f32),
      scratch_types=[
          pltpu.VMEM((SUB,), jnp.int32),
          pltpu.VMEM((SUB,), jnp.int32),
          pltpu.VMEM((CH,), jnp.int32),
          pltpu.VMEM((CH,), jnp.int32),
          pltpu.VMEM((CH, D), _f32),
          pltpu.VMEM((CH, D), _f32),
          pltpu.VMEM((40, D), _f32),
          pltpu.VMEM_SHARED((NP, D), _f32),
          pltpu.SemaphoreType.DMA,
          pltpu.SemaphoreType.DMA,
      ],
  )
  def body(tmp_hbm, src_hbm, dst_hbm, out_hbm, sbuf, dbuf, didx0, didx1,
           rows0, rows1, zbuf, agg, sem0, sem1):
    c = lax.axis_index("c")
    s = lax.axis_index("s")
    rpw = NP // NS

    def zrow(r, _):
      for q in range(D // 16):
        zbuf[r, pl.ds(q * 16, 16)] = jnp.zeros((16,), _f32)
      return 0

    lax.fori_loop(0, 40, zrow, 0)

    base = c * EPC + s * SUB
    ne = jnp.where(s < NS - 1, SUB, EPC - (NS - 1) * SUB)
    nk = ne // CH

    for g in range(NG):
      goff = pl.multiple_of(g * E + base, 8)
      pltpu.sync_copy(src_hbm.at[pl.ds(goff, SUB)], sbuf)
      pltpu.sync_copy(dst_hbm.at[pl.ds(goff, SUB)], dbuf)
      for q in range(rpw // 40):
        pltpu.sync_copy(zbuf, agg.at[pl.ds(s * rpw + q * 40, 40)])
      plsc.subcore_barrier()

      def gather(k, rows, sem):
        pltpu.async_copy(tmp_hbm.at[g].at[sbuf.at[pl.ds(k * CH, CH)]],
                         rows, sem)

      def gwait(rows, sem):
        pltpu.make_async_copy(tmp_hbm.at[g].at[sbuf.at[pl.ds(0, CH)]],
                              rows, sem).wait()

      def dcopy(k, didx):
        for q in range(CH // 16):
          didx[pl.ds(q * 16, 16)] = dbuf[pl.ds(k * CH + q * 16, 16)]

      gather(0, rows0, sem0)

      def pair(j, _):
        a = 2 * j

        @pl.when(a + 1 < nk)
        def _():
          gather(a + 1, rows1, sem1)

        dcopy(a, didx0)
        gwait(rows0, sem0)
        pltpu.sync_copy(rows0, agg.at[didx0], add=True)

        @pl.when(a + 2 < nk)
        def _():
          gather(a + 2, rows0, sem0)

        @pl.when(a + 1 < nk)
        def _():
          dcopy(a + 1, didx1)
          gwait(rows1, sem1)
          pltpu.sync_copy(rows1, agg.at[didx1], add=True)

        return 0

      lax.fori_loop(0, (nk + 1) // 2, pair, 0)
      plsc.subcore_barrier()
      for q in range(rpw // CH):
        r0 = s * rpw + q * CH
        pltpu.sync_copy(agg.at[pl.ds(r0, CH)], rows0)
        pltpu.sync_copy(rows0, out_hbm.at[c, g, pl.ds(r0, CH)])
      plsc.subcore_barrier()

  return body(tmp_all, src_all, dst_all)


def _tc_pre(degp, gfe0, ofe0, W0):
  grid = (NP // BN,)

  def body(deg_ref, gfe_ref, ofe_ref, w_ref, tmp_ref, dinv_ref):
    deg = deg_ref[...]
    dinv = lax.rsqrt(deg[0] + deg[1] + 1.0)
    w = w_ref[...]
    for g in range(NG):
      x = gfe_ref[g] if g < C else ofe_ref[...]
      h = jnp.dot(x, w, preferred_element_type=_f32)
      tmp_ref[g, :, :] = h * dinv[g][:, None]
    dinv_ref[...] = dinv

  return pl.pallas_call(
      body,
      grid=grid,
      in_specs=[
          pl.BlockSpec((NC, NG, BN), lambda i: (0, 0, i)),
          pl.BlockSpec((C, BN, D), lambda i: (0, i, 0)),
          pl.BlockSpec((BN, D), lambda i: (i, 0)),
          pl.BlockSpec((D, D), lambda i: (0, 0)),
      ],
      out_specs=[
          pl.BlockSpec((NG, BN, D), lambda i: (0, i, 0)),
          pl.BlockSpec((NG, BN), lambda i: (0, i)),
      ],
      out_shape=[
          jax.ShapeDtypeStruct((NG, NP, D), _f32),
          jax.ShapeDtypeStruct((NG, NP), _f32),
      ],
  )(degp, gfe0, ofe0, W0)


def _attention_stage(tmp, agg, dinv, label_emb, Wq, bq, b, Wk, bk, Wv, bv,
                     W_next=None):
  final = W_next is None
  grid = (NP // BN,)
  inv_sqrt_d = 1.0 / math.sqrt(D)

  def body(tmp_ref, agg_ref, dinv_ref, lemb_ref, wq_ref, bq_ref, b_ref,
           wk_ref, bk_ref, wv_ref, bv_ref, *rest):
    if final:
      gfe_ref, ofe_ref = rest
    else:
      (wn_ref, out_ref) = rest
    lq = jnp.dot(lemb_ref[...], wq_ref[...],
                 preferred_element_type=_f32) + bq_ref[...]
    dinv = dinv_ref[...]
    bias = b_ref[...]
    conv = []
    for g in range(NG):
      cg = dinv[g][:, None] * (agg_ref[0, g] + agg_ref[1, g] + tmp_ref[g])
      conv.append(cg + bias)
    wk = wk_ref[...]
    wv = wv_ref[...]
    bk = bk_ref[...]
    bv = bv_ref[...]
    scores = []
    vs = []
    for g in range(C):
      kg = jnp.dot(conv[g], wk, preferred_element_type=_f32) + bk
      vs.append(jnp.dot(conv[g], wv, preferred_element_type=_f32) + bv)
      scores.append([
          jnp.sum(kg * lq[a][None, :], axis=1) * inv_sqrt_d for a in range(C)
      ])
    outs = []
    for a in range(C):
      m = scores[0][a]
      for g in range(1, C):
        m = jnp.maximum(m, scores[g][a])
      es = [jnp.exp(scores[g][a] - m) for g in range(C)]
      z = es[0] + es[1] + es[2] + es[3]
      o = (es[0] / z)[:, None] * vs[0]
      for g in range(1, C):
        o = o + (es[g] / z)[:, None] * vs[g]
      outs.append(o)
    if final:
      for a in range(C):
        gfe_ref[a, :, :] = outs[a]
      ofe_ref[...] = conv[C]
    else:
      wn = wn_ref[...]
      for g in range(NG):
        x2 = jnp.maximum(outs[g] if g < C else conv[C], 0.0)
        h2 = jnp.dot(x2, wn, preferred_element_type=_f32)
        out_ref[g, :, :] = h2 * dinv[g][:, None]

  full = lambda shape: pl.BlockSpec(shape, lambda i: tuple(0 for _ in shape))
  in_specs = [
      pl.BlockSpec((NG, BN, D), lambda i: (0, i, 0)),
      pl.BlockSpec((NC, NG, BN, D), lambda i: (0, 0, i, 0)),
      pl.BlockSpec((NG, BN), lambda i: (0, i)),
      full((C, D)), full((D, D)), full((D,)), full((D,)),
      full((D, D)), full((D,)), full((D, D)), full((D,)),
  ]
  args = [tmp, agg, dinv, label_emb, Wq, bq, b, Wk, bk, Wv, bv]
  if final:
    out_specs = [
        pl.BlockSpec((C, BN, D), lambda i: (0, i, 0)),
        pl.BlockSpec((BN, D), lambda i: (i, 0)),
    ]
    out_shape = [
        jax.ShapeDtypeStruct((C, NP, D), _f32),
        jax.ShapeDtypeStruct((NP, D), _f32),
    ]
  else:
    in_specs.append(full((D, D)))
    args.append(W_next)
    out_specs = [pl.BlockSpec((NG, BN, D), lambda i: (0, i, 0))]
    out_shape = [jax.ShapeDtypeStruct((NG, NP, D), _f32)]
  return pl.pallas_call(
      body, grid=grid, in_specs=in_specs, out_specs=out_specs,
      out_shape=out_shape)(*args)


def kernel(graph_feat_emb, ori_feat_emb, label_emb, edge_index_label,
           edge_index_ori, W0, b0, W1, b1, Wq, bq, Wk, bk, Wv, bv):
  gfe0 = jnp.pad(graph_feat_emb, ((0, 0), (0, NP - N), (0, 0)))
  ofe0 = jnp.pad(ori_feat_emb, ((0, NP - N), (0, 0)))
  eil = edge_index_label.astype(jnp.int32)
  eio = edge_index_ori.astype(jnp.int32)
  src_all = jnp.pad(
      jnp.concatenate([eil[:, 0, :], eio[0:1]], axis=0).reshape(-1), (0, 2048))
  dst_all = jnp.pad(
      jnp.concatenate([eil[:, 1, :], eio[1:2]], axis=0).reshape(-1), (0, 2048))

  degp = _deg_counts(dst_all).reshape(NC, NG, NP)
  tmp1, dinv = _tc_pre(degp, gfe0, ofe0, W0)
  agg1 = _edge_aggregate(tmp1, src_all, dst_all)
  (tmp2,) = _attention_stage(tmp1, agg1, dinv, label_emb, Wq, bq, b0,
                             Wk, bk, Wv, bv, W_next=W1)
  agg2 = _edge_aggregate(tmp2, src_all, dst_all)
  gfe_f, ofe_f = _attention_stage(tmp2, agg2, dinv, label_emb, Wq, bq, b1,
                                  Wk, bk, Wv, bv)
  return gfe_f[:, :N, :], ofe_f[:N, :]


# speedup vs baseline: 4.3473x; 1.0091x over previous
import functools
import math

import jax
import jax.numpy as jnp
from jax import lax
from jax.experimental import pallas as pl
from jax.experimental.pallas import tpu as pltpu
from jax.experimental.pallas import tpu_sc as plsc

N = 10000
E = 160000
C = 4
D = 128
NG = 5
NP = 10240
BN = 512
NC = 2
NS = 16
CH = 128
EPC = E // NC
SUB = 5120

_f32 = jnp.float32


def _deg_counts(dst_all):
  mesh = plsc.VectorSubcoreMesh(core_axis_name="c", subcore_axis_name="s")

  @functools.partial(
      pl.kernel,
      mesh=mesh,
      out_type=jax.ShapeDtypeStruct((NC * NG * NP,), _f32),
      scratch_types=[
          pltpu.VMEM((SUB,), jnp.int32),
          pltpu.VMEM((CH,), jnp.int32),
          pltpu.VMEM((CH,), _f32),
          pltpu.VMEM((1600,), _f32),
          pltpu.VMEM_SHARED((NG * NP,), _f32),
      ],
  )
  def body(dst_hbm, out_hbm, dbuf, gidx, ones, zb, deg):
    c = lax.axis_index("c")
    s = lax.axis_index("s")
    for q in range(CH // 16):
      ones[pl.ds(q * 16, 16)] = jnp.full((16,), 1.0, _f32)

    def zrow(j, _):
      zb[pl.ds(j * 16, 16)] = jnp.zeros((16,), _f32)
      return 0

    lax.fori_loop(0, 100, zrow, 0)
    for t in range(2):
      pltpu.sync_copy(zb, deg.at[pl.ds(s * 3200 + t * 1600, 1600)])
    plsc.subcore_barrier()

    base = c * EPC + s * SUB
    ne = jnp.where(s < NS - 1, SUB, EPC - (NS - 1) * SUB)
    nk = ne // CH
    for g in range(NG):
      pltpu.sync_copy(
          dst_hbm.at[pl.ds(pl.multiple_of(g * E + base, 8), SUB)], dbuf)

      def chunk(k, _):
        for q in range(CH // 16):
          gidx[pl.ds(q * 16, 16)] = dbuf[pl.ds(k * CH + q * 16, 16)] + g * NP
        pltpu.sync_copy(ones, deg.at[gidx], add=True)
        return 0

      lax.fori_loop(0, nk, chunk, 0)
    plsc.subcore_barrier()
    for t in range(2):
      o = s * 3200 + t * 1600
      pltpu.sync_copy(deg.at[pl.ds(o, 1600)], zb)
      pltpu.sync_copy(zb, out_hbm.at[pl.ds(c * (NG * NP) + o, 1600)])

  return body(dst_all)


def _edge_aggregate(tmp_all, src_all, dst_all):
  mesh = plsc.VectorSubcoreMesh(core_axis_name="c", subcore_axis_name="s")

  @functools.partial(
      pl.kernel,
      mesh=mesh,
      out_type=jax.ShapeDtypeStruct((NC, NG, NP, D), _f32),
      scratch_types=[
          pltpu.VMEM((SUB,), jnp.int32),
          pltpu.VMEM((SUB,), jnp.int32),
          pltpu.VMEM((CH,), jnp.int32),
          pltpu.VMEM((CH,), jnp.int32),
          pltpu.VMEM((CH, D), _f32),
          pltpu.VMEM((CH, D), _f32),
          pltpu.VMEM((40, D), _f32),
          pltpu.VMEM_SHARED((NP, D), _f32),
          pltpu.SemaphoreType.DMA,
          pltpu.SemaphoreType.DMA,
      ],
  )
  def body(tmp_hbm, src_hbm, dst_hbm, out_hbm, sbuf, dbuf, didx0, didx1,
           rows0, rows1, zbuf, agg, sem0, sem1):
    c = lax.axis_index("c")
    s = lax.axis_index("s")
    rpw = NP // NS

    def zrow(r, _):
      for q in range(D // 16):
        zbuf[r, pl.ds(q * 16, 16)] = jnp.zeros((16,), _f32)
      return 0

    lax.fori_loop(0, 40, zrow, 0)

    base = c * EPC + s * SUB
    ne = jnp.where(s < NS - 1, SUB, EPC - (NS - 1) * SUB)
    nk = ne // CH

    for g in range(NG):
      goff = pl.multiple_of(g * E + base, 8)
      pltpu.sync_copy(src_hbm.at[pl.ds(goff, SUB)], sbuf)
      pltpu.sync_copy(dst_hbm.at[pl.ds(goff, SUB)], dbuf)
      zd = [
          pltpu.async_copy(zbuf, agg.at[pl.ds(s * rpw + q * 40, 40)], sem0)
          for q in range(rpw // 40)
      ]
      for d in zd:
        d.wait()
      plsc.subcore_barrier()

      def gather(k, rows, sem):
        pltpu.async_copy(tmp_hbm.at[g].at[sbuf.at[pl.ds(k * CH, CH)]],
                         rows, sem)

      def gwait(rows, sem):
        pltpu.make_async_copy(tmp_hbm.at[g].at[sbuf.at[pl.ds(0, CH)]],
                              rows, sem).wait()

      def dcopy(k, didx):
        for q in range(CH // 16):
          didx[pl.ds(q * 16, 16)] = dbuf[pl.ds(k * CH + q * 16, 16)]

      gather(0, rows0, sem0)

      def pair(j, _):
        a = 2 * j

        @pl.when(a + 1 < nk)
        def _():
          gather(a + 1, rows1, sem1)

        dcopy(a, didx0)
        gwait(rows0, sem0)
        pltpu.sync_copy(rows0, agg.at[didx0], add=True)

        @pl.when(a + 2 < nk)
        def _():
          gather(a + 2, rows0, sem0)

        @pl.when(a + 1 < nk)
        def _():
          dcopy(a + 1, didx1)
          gwait(rows1, sem1)
          pltpu.sync_copy(rows1, agg.at[didx1], add=True)

        return 0

      lax.fori_loop(0, (nk + 1) // 2, pair, 0)
      plsc.subcore_barrier()
      wd = [
          pltpu.async_copy(agg.at[pl.ds(s * rpw + q * CH, CH)],
                           out_hbm.at[c, g, pl.ds(s * rpw + q * CH, CH)],
                           sem1)
          for q in range(rpw // CH)
      ]
      for d in wd:
        d.wait()
      plsc.subcore_barrier()

  return body(tmp_all, src_all, dst_all)


def _tc_pre(degp, gfe0, ofe0, W0):
  grid = (NP // BN,)

  def body(deg_ref, gfe_ref, ofe_ref, w_ref, tmp_ref, dinv_ref):
    deg = deg_ref[...]
    dinv = lax.rsqrt(deg[0] + deg[1] + 1.0)
    w = w_ref[...]
    for g in range(NG):
      x = gfe_ref[g] if g < C else ofe_ref[...]
      h = jnp.dot(x, w, preferred_element_type=_f32)
      tmp_ref[g, :, :] = h * dinv[g][:, None]
    dinv_ref[...] = dinv

  return pl.pallas_call(
      body,
      grid=grid,
      in_specs=[
          pl.BlockSpec((NC, NG, BN), lambda i: (0, 0, i)),
          pl.BlockSpec((C, BN, D), lambda i: (0, i, 0)),
          pl.BlockSpec((BN, D), lambda i: (i, 0)),
          pl.BlockSpec((D, D), lambda i: (0, 0)),
      ],
      out_specs=[
          pl.BlockSpec((NG, BN, D), lambda i: (0, i, 0)),
          pl.BlockSpec((NG, BN), lambda i: (0, i)),
      ],
      out_shape=[
          jax.ShapeDtypeStruct((NG, NP, D), _f32),
          jax.ShapeDtypeStruct((NG, NP), _f32),
      ],
  )(degp, gfe0, ofe0, W0)


def _attention_stage(tmp, agg, dinv, label_emb, Wq, bq, b, Wk, bk, Wv, bv,
                     W_next=None):
  final = W_next is None
  grid = (NP // BN,)
  inv_sqrt_d = 1.0 / math.sqrt(D)

  def body(tmp_ref, agg_ref, dinv_ref, lemb_ref, wq_ref, bq_ref, b_ref,
           wk_ref, bk_ref, wv_ref, bv_ref, *rest):
    if final:
      gfe_ref, ofe_ref = rest
    else:
      (wn_ref, out_ref) = rest
    lq = jnp.dot(lemb_ref[...], wq_ref[...],
                 preferred_element_type=_f32) + bq_ref[...]
    dinv = dinv_ref[...]
    bias = b_ref[...]
    conv = []
    for g in range(NG):
      cg = dinv[g][:, None] * (agg_ref[0, g] + agg_ref[1, g] + tmp_ref[g])
      conv.append(cg + bias)
    wk = wk_ref[...]
    wv = wv_ref[...]
    bk = bk_ref[...]
    bv = bv_ref[...]
    scores = []
    vs = []
    for g in range(C):
      kg = jnp.dot(conv[g], wk, preferred_element_type=_f32) + bk
      vs.append(jnp.dot(conv[g], wv, preferred_element_type=_f32) + bv)
      scores.append([
          jnp.sum(kg * lq[a][None, :], axis=1) * inv_sqrt_d for a in range(C)
      ])
    outs = []
    for a in range(C):
      m = scores[0][a]
      for g in range(1, C):
        m = jnp.maximum(m, scores[g][a])
      es = [jnp.exp(scores[g][a] - m) for g in range(C)]
      z = es[0] + es[1] + es[2] + es[3]
      o = (es[0] / z)[:, None] * vs[0]
      for g in range(1, C):
        o = o + (es[g] / z)[:, None] * vs[g]
      outs.append(o)
    if final:
      for a in range(C):
        gfe_ref[a, :, :] = outs[a]
      ofe_ref[...] = conv[C]
    else:
      wn = wn_ref[...]
      for g in range(NG):
        x2 = jnp.maximum(outs[g] if g < C else conv[C], 0.0)
        h2 = jnp.dot(x2, wn, preferred_element_type=_f32)
        out_ref[g, :, :] = h2 * dinv[g][:, None]

  full = lambda shape: pl.BlockSpec(shape, lambda i: tuple(0 for _ in shape))
  in_specs = [
      pl.BlockSpec((NG, BN, D), lambda i: (0, i, 0)),
      pl.BlockSpec((NC, NG, BN, D), lambda i: (0, 0, i, 0)),
      pl.BlockSpec((NG, BN), lambda i: (0, i)),
      full((C, D)), full((D, D)), full((D,)), full((D,)),
      full((D, D)), full((D,)), full((D, D)), full((D,)),
  ]
  args = [tmp, agg, dinv, label_emb, Wq, bq, b, Wk, bk, Wv, bv]
  if final:
    out_specs = [
        pl.BlockSpec((C, BN, D), lambda i: (0, i, 0)),
        pl.BlockSpec((BN, D), lambda i: (i, 0)),
    ]
    out_shape = [
        jax.ShapeDtypeStruct((C, NP, D), _f32),
        jax.ShapeDtypeStruct((NP, D), _f32),
    ]
  else:
    in_specs.append(full((D, D)))
    args.append(W_next)
    out_specs = [pl.BlockSpec((NG, BN, D), lambda i: (0, i, 0))]
    out_shape = [jax.ShapeDtypeStruct((NG, NP, D), _f32)]
  return pl.pallas_call(
      body, grid=grid, in_specs=in_specs, out_specs=out_specs,
      out_shape=out_shape)(*args)


def kernel(graph_feat_emb, ori_feat_emb, label_emb, edge_index_label,
           edge_index_ori, W0, b0, W1, b1, Wq, bq, Wk, bk, Wv, bv):
  gfe0 = jnp.pad(graph_feat_emb, ((0, 0), (0, NP - N), (0, 0)))
  ofe0 = jnp.pad(ori_feat_emb, ((0, NP - N), (0, 0)))
  eil = edge_index_label.astype(jnp.int32)
  eio = edge_index_ori.astype(jnp.int32)
  src_all = jnp.pad(
      jnp.concatenate([eil[:, 0, :], eio[0:1]], axis=0).reshape(-1), (0, 2048))
  dst_all = jnp.pad(
      jnp.concatenate([eil[:, 1, :], eio[1:2]], axis=0).reshape(-1), (0, 2048))

  degp = _deg_counts(dst_all).reshape(NC, NG, NP)
  tmp1, dinv = _tc_pre(degp, gfe0, ofe0, W0)
  agg1 = _edge_aggregate(tmp1, src_all, dst_all)
  (tmp2,) = _attention_stage(tmp1, agg1, dinv, label_emb, Wq, bq, b0,
                             Wk, bk, Wv, bv, W_next=W1)
  agg2 = _edge_aggregate(tmp2, src_all, dst_all)
  gfe_f, ofe_f = _attention_stage(tmp2, agg2, dinv, label_emb, Wq, bq, b1,
                                  Wk, bk, Wv, bv)
  return gfe_f[:, :N, :], ofe_f[:N, :]


# speedup vs baseline: 4.4722x; 1.0287x over previous
import functools
import math

import jax
import jax.numpy as jnp
from jax import lax
from jax.experimental import pallas as pl
from jax.experimental.pallas import tpu as pltpu
from jax.experimental.pallas import tpu_sc as plsc

N = 10000
E = 160000
C = 4
D = 128
NG = 5
NP = 10240
BN = 512
NC = 2
NS = 16
CH = 128
EPC = E // NC
SUB = 5120

_f32 = jnp.float32


def _deg_counts(dst_all):
  mesh = plsc.VectorSubcoreMesh(core_axis_name="c", subcore_axis_name="s")

  @functools.partial(
      pl.kernel,
      mesh=mesh,
      out_type=jax.ShapeDtypeStruct((NC * NG * NP,), _f32),
      scratch_types=[
          pltpu.VMEM((SUB,), jnp.int32),
          pltpu.VMEM((CH,), jnp.int32),
          pltpu.VMEM((CH,), _f32),
          pltpu.VMEM((1600,), _f32),
          pltpu.VMEM_SHARED((NG * NP,), _f32),
      ],
  )
  def body(dst_hbm, out_hbm, dbuf, gidx, ones, zb, deg):
    c = lax.axis_index("c")
    s = lax.axis_index("s")
    for q in range(CH // 16):
      ones[pl.ds(q * 16, 16)] = jnp.full((16,), 1.0, _f32)

    def zrow(j, _):
      zb[pl.ds(j * 16, 16)] = jnp.zeros((16,), _f32)
      return 0

    lax.fori_loop(0, 100, zrow, 0)
    for t in range(2):
      pltpu.sync_copy(zb, deg.at[pl.ds(s * 3200 + t * 1600, 1600)])
    plsc.subcore_barrier()

    base = c * EPC + s * SUB
    ne = jnp.where(s < NS - 1, SUB, EPC - (NS - 1) * SUB)
    nk = ne // CH
    for g in range(NG):
      pltpu.sync_copy(
          dst_hbm.at[pl.ds(pl.multiple_of(g * E + base, 8), SUB)], dbuf)

      def chunk(k, _):
        for q in range(CH // 16):
          gidx[pl.ds(q * 16, 16)] = dbuf[pl.ds(k * CH + q * 16, 16)] + g * NP
        pltpu.sync_copy(ones, deg.at[gidx], add=True)
        return 0

      lax.fori_loop(0, nk, chunk, 0)
    plsc.subcore_barrier()
    for t in range(2):
      o = s * 3200 + t * 1600
      pltpu.sync_copy(deg.at[pl.ds(o, 1600)], zb)
      pltpu.sync_copy(zb, out_hbm.at[pl.ds(c * (NG * NP) + o, 1600)])

  return body(dst_all)


def _edge_aggregate(tmp_all, src_all, dst_all):
  mesh = plsc.VectorSubcoreMesh(core_axis_name="c", subcore_axis_name="s")

  @functools.partial(
      pl.kernel,
      mesh=mesh,
      out_type=jax.ShapeDtypeStruct((NC, NG, NP, D), _f32),
      scratch_types=[
          pltpu.VMEM((SUB,), jnp.int32),
          pltpu.VMEM((SUB,), jnp.int32),
          pltpu.VMEM((CH,), jnp.int32),
          pltpu.VMEM((CH,), jnp.int32),
          pltpu.VMEM((CH, D), _f32),
          pltpu.VMEM((CH, D), _f32),
          pltpu.VMEM((40, D), _f32),
          pltpu.VMEM_SHARED((NP, D), _f32),
          pltpu.SemaphoreType.DMA,
          pltpu.SemaphoreType.DMA,
      ],
  )
  def body(tmp_hbm, src_hbm, dst_hbm, out_hbm, sbuf, dbuf, didx0, didx1,
           rows0, rows1, zbuf, agg, sem0, sem1):
    c = lax.axis_index("c")
    s = lax.axis_index("s")
    rpw = NP // NS

    def zrow(r, _):
      for q in range(D // 16):
        zbuf[r, pl.ds(q * 16, 16)] = jnp.zeros((16,), _f32)
      return 0

    lax.fori_loop(0, 40, zrow, 0)

    base = c * EPC + s * SUB
    ne = jnp.where(s < NS - 1, SUB, EPC - (NS - 1) * SUB)
    nk = ne // CH

    zd = [
        pltpu.async_copy(zbuf, agg.at[pl.ds(s * rpw + q * 40, 40)], sem0)
        for q in range(rpw // 40)
    ]
    for d in zd:
      d.wait()

    for g in range(NG):
      goff = pl.multiple_of(g * E + base, 8)
      pltpu.sync_copy(src_hbm.at[pl.ds(goff, SUB)], sbuf)
      pltpu.sync_copy(dst_hbm.at[pl.ds(goff, SUB)], dbuf)
      plsc.subcore_barrier()

      def gather(k, rows, sem):
        pltpu.async_copy(tmp_hbm.at[g].at[sbuf.at[pl.ds(k * CH, CH)]],
                         rows, sem)

      def gwait(rows, sem):
        pltpu.make_async_copy(tmp_hbm.at[g].at[sbuf.at[pl.ds(0, CH)]],
                              rows, sem).wait()

      def dcopy(k, didx):
        for q in range(CH // 16):
          didx[pl.ds(q * 16, 16)] = dbuf[pl.ds(k * CH + q * 16, 16)]

      gather(0, rows0, sem0)

      def pair(j, _):
        a = 2 * j

        @pl.when(a + 1 < nk)
        def _():
          gather(a + 1, rows1, sem1)

        dcopy(a, didx0)
        gwait(rows0, sem0)
        pltpu.sync_copy(rows0, agg.at[didx0], add=True)

        @pl.when(a + 2 < nk)
        def _():
          gather(a + 2, rows0, sem0)

        @pl.when(a + 1 < nk)
        def _():
          dcopy(a + 1, didx1)
          gwait(rows1, sem1)
          pltpu.sync_copy(rows1, agg.at[didx1], add=True)

        return 0

      lax.fori_loop(0, (nk + 1) // 2, pair, 0)
      plsc.subcore_barrier()
      wd = [
          pltpu.async_copy(agg.at[pl.ds(s * rpw + q * CH, CH)],
                           out_hbm.at[c, g, pl.ds(s * rpw + q * CH, CH)],
                           sem1)
          for q in range(rpw // CH)
      ]
      for d in wd:
        d.wait()
      plsc.subcore_barrier()

  return body(tmp_all, src_all, dst_all)


def _tc_pre(degp, gfe0, ofe0, W0):
  grid = (NP // BN,)

  def body(deg_ref, gfe_ref, ofe_ref, w_ref, tmp_ref, dinv_ref):
    deg = deg_ref[...]
    dinv = lax.rsqrt(deg[0] + deg[1] + 1.0)
    w = w_ref[...]
    for g in range(NG):
      x = gfe_ref[g] if g < C else ofe_ref[...]
      h = jnp.dot(x, w, preferred_element_type=_f32)
      tmp_ref[g, :, :] = h * dinv[g][:, None]
    dinv_ref[...] = dinv

  return pl.pallas_call(
      body,
      grid=grid,
      in_specs=[
          pl.BlockSpec((NC, NG, BN), lambda i: (0, 0, i)),
          pl.BlockSpec((C, BN, D), lambda i: (0, i, 0)),
          pl.BlockSpec((BN, D), lambda i: (i, 0)),
          pl.BlockSpec((D, D), lambda i: (0, 0)),
      ],
      out_specs=[
          pl.BlockSpec((NG, BN, D), lambda i: (0, i, 0)),
          pl.BlockSpec((NG, BN), lambda i: (0, i)),
      ],
      out_shape=[
          jax.ShapeDtypeStruct((NG, NP, D), _f32),
          jax.ShapeDtypeStruct((NG, NP), _f32),
      ],
  )(degp, gfe0, ofe0, W0)


def _attention_stage(tmp, agg, dinv, label_emb, Wq, bq, b, Wk, bk, Wv, bv,
                     W_next=None):
  final = W_next is None
  grid = (NP // BN,)
  inv_sqrt_d = 1.0 / math.sqrt(D)

  def body(tmp_ref, agg_ref, dinv_ref, lemb_ref, wq_ref, bq_ref, b_ref,
           wk_ref, bk_ref, wv_ref, bv_ref, *rest):
    if final:
      gfe_ref, ofe_ref = rest
    else:
      (wn_ref, out_ref) = rest
    lq = jnp.dot(lemb_ref[...], wq_ref[...],
                 preferred_element_type=_f32) + bq_ref[...]
    dinv = dinv_ref[...]
    bias = b_ref[...]
    conv = []
    prev = None
    for g in range(NG):
      cum = agg_ref[0, g] + agg_ref[1, g]
      delta = cum if prev is None else cum - prev
      prev = cum
      cg = dinv[g][:, None] * (delta + tmp_ref[g])
      conv.append(cg + bias)
    wk = wk_ref[...]
    wv = wv_ref[...]
    bk = bk_ref[...]
    bv = bv_ref[...]
    scores = []
    vs = []
    for g in range(C):
      kg = jnp.dot(conv[g], wk, preferred_element_type=_f32) + bk
      vs.append(jnp.dot(conv[g], wv, preferred_element_type=_f32) + bv)
      scores.append([
          jnp.sum(kg * lq[a][None, :], axis=1) * inv_sqrt_d for a in range(C)
      ])
    outs = []
    for a in range(C):
      m = scores[0][a]
      for g in range(1, C):
        m = jnp.maximum(m, scores[g][a])
      es = [jnp.exp(scores[g][a] - m) for g in range(C)]
      z = es[0] + es[1] + es[2] + es[3]
      o = (es[0] / z)[:, None] * vs[0]
      for g in range(1, C):
        o = o + (es[g] / z)[:, None] * vs[g]
      outs.append(o)
    if final:
      for a in range(C):
        gfe_ref[a, :, :] = outs[a]
      ofe_ref[...] = conv[C]
    else:
      wn = wn_ref[...]
      for g in range(NG):
        x2 = jnp.maximum(outs[g] if g < C else conv[C], 0.0)
        h2 = jnp.dot(x2, wn, preferred_element_type=_f32)
        out_ref[g, :, :] = h2 * dinv[g][:, None]

  full = lambda shape: pl.BlockSpec(shape, lambda i: tuple(0 for _ in shape))
  in_specs = [
      pl.BlockSpec((NG, BN, D), lambda i: (0, i, 0)),
      pl.BlockSpec((NC, NG, BN, D), lambda i: (0, 0, i, 0)),
      pl.BlockSpec((NG, BN), lambda i: (0, i)),
      full((C, D)), full((D, D)), full((D,)), full((D,)),
      full((D, D)), full((D,)), full((D, D)), full((D,)),
  ]
  args = [tmp, agg, dinv, label_emb, Wq, bq, b, Wk, bk, Wv, bv]
  if final:
    out_specs = [
        pl.BlockSpec((C, BN, D), lambda i: (0, i, 0)),
        pl.BlockSpec((BN, D), lambda i: (i, 0)),
    ]
    out_shape = [
        jax.ShapeDtypeStruct((C, NP, D), _f32),
        jax.ShapeDtypeStruct((NP, D), _f32),
    ]
  else:
    in_specs.append(full((D, D)))
    args.append(W_next)
    out_specs = [pl.BlockSpec((NG, BN, D), lambda i: (0, i, 0))]
    out_shape = [jax.ShapeDtypeStruct((NG, NP, D), _f32)]
  return pl.pallas_call(
      body, grid=grid, in_specs=in_specs, out_specs=out_specs,
      out_shape=out_shape)(*args)


def kernel(graph_feat_emb, ori_feat_emb, label_emb, edge_index_label,
           edge_index_ori, W0, b0, W1, b1, Wq, bq, Wk, bk, Wv, bv):
  gfe0 = jnp.pad(graph_feat_emb, ((0, 0), (0, NP - N), (0, 0)))
  ofe0 = jnp.pad(ori_feat_emb, ((0, NP - N), (0, 0)))
  eil = edge_index_label.astype(jnp.int32)
  eio = edge_index_ori.astype(jnp.int32)
  src_all = jnp.pad(
      jnp.concatenate([eil[:, 0, :], eio[0:1]], axis=0).reshape(-1), (0, 2048))
  dst_all = jnp.pad(
      jnp.concatenate([eil[:, 1, :], eio[1:2]], axis=0).reshape(-1), (0, 2048))

  degp = _deg_counts(dst_all).reshape(NC, NG, NP)
  tmp1, dinv = _tc_pre(degp, gfe0, ofe0, W0)
  agg1 = _edge_aggregate(tmp1, src_all, dst_all)
  (tmp2,) = _attention_stage(tmp1, agg1, dinv, label_emb, Wq, bq, b0,
                             Wk, bk, Wv, bv, W_next=W1)
  agg2 = _edge_aggregate(tmp2, src_all, dst_all)
  gfe_f, ofe_f = _attention_stage(tmp2, agg2, dinv, label_emb, Wq, bq, b1,
                                  Wk, bk, Wv, bv)
  return gfe_f[:, :N, :], ofe_f[:N, :]


# speedup vs baseline: 4.5905x; 1.0265x over previous
import functools
import math

import jax
import jax.numpy as jnp
from jax import lax
from jax.experimental import pallas as pl
from jax.experimental.pallas import tpu as pltpu
from jax.experimental.pallas import tpu_sc as plsc

N = 10000
E = 160000
C = 4
D = 128
NG = 5
NP = 10240
BN = 512
NC = 2
NS = 16
CH = 128
EPC = E // NC
SUB = 5120

_f32 = jnp.float32


def _deg_counts(dst_all):
  mesh = plsc.VectorSubcoreMesh(core_axis_name="c", subcore_axis_name="s")

  @functools.partial(
      pl.kernel,
      mesh=mesh,
      out_type=jax.ShapeDtypeStruct((NC * NG * NP,), _f32),
      scratch_types=[
          pltpu.VMEM((SUB,), jnp.int32),
          pltpu.VMEM((CH,), jnp.int32),
          pltpu.VMEM((CH,), _f32),
          pltpu.VMEM((1600,), _f32),
          pltpu.VMEM_SHARED((NG * NP,), _f32),
      ],
  )
  def body(dst_hbm, out_hbm, dbuf, gidx, ones, zb, deg):
    c = lax.axis_index("c")
    s = lax.axis_index("s")
    for q in range(CH // 16):
      ones[pl.ds(q * 16, 16)] = jnp.full((16,), 1.0, _f32)

    def zrow(j, _):
      zb[pl.ds(j * 16, 16)] = jnp.zeros((16,), _f32)
      return 0

    lax.fori_loop(0, 100, zrow, 0)
    for t in range(2):
      pltpu.sync_copy(zb, deg.at[pl.ds(s * 3200 + t * 1600, 1600)])
    plsc.subcore_barrier()

    base = c * EPC + s * SUB
    ne = jnp.where(s < NS - 1, SUB, EPC - (NS - 1) * SUB)
    nk = ne // CH
    for g in range(NG):
      pltpu.sync_copy(
          dst_hbm.at[pl.ds(pl.multiple_of(g * E + base, 8), SUB)], dbuf)

      def chunk(k, _):
        for q in range(CH // 16):
          gidx[pl.ds(q * 16, 16)] = dbuf[pl.ds(k * CH + q * 16, 16)] + g * NP
        pltpu.sync_copy(ones, deg.at[gidx], add=True)
        return 0

      lax.fori_loop(0, nk, chunk, 0)
    plsc.subcore_barrier()
    for t in range(2):
      o = s * 3200 + t * 1600
      pltpu.sync_copy(deg.at[pl.ds(o, 1600)], zb)
      pltpu.sync_copy(zb, out_hbm.at[pl.ds(c * (NG * NP) + o, 1600)])

  return body(dst_all)


def _edge_aggregate(tmp_all, src_all, dst_all):
  mesh = plsc.VectorSubcoreMesh(core_axis_name="c", subcore_axis_name="s")

  @functools.partial(
      pl.kernel,
      mesh=mesh,
      out_type=jax.ShapeDtypeStruct((NC, NG, NP, D), _f32),
      scratch_types=[
          pltpu.VMEM((SUB,), jnp.int32),
          pltpu.VMEM((SUB,), jnp.int32),
          pltpu.VMEM((CH,), jnp.int32),
          pltpu.VMEM((CH,), jnp.int32),
          pltpu.VMEM((CH, D), _f32),
          pltpu.VMEM((CH, D), _f32),
          pltpu.VMEM((40, D), _f32),
          pltpu.VMEM_SHARED((NP, D), _f32),
          pltpu.SemaphoreType.DMA,
          pltpu.SemaphoreType.DMA,
      ],
  )
  def body(tmp_hbm, src_hbm, dst_hbm, out_hbm, sbuf, dbuf, didx0, didx1,
           rows0, rows1, zbuf, agg, sem0, sem1):
    c = lax.axis_index("c")
    s = lax.axis_index("s")
    rpw = NP // NS

    def zrow(r, _):
      for q in range(D // 16):
        zbuf[r, pl.ds(q * 16, 16)] = jnp.zeros((16,), _f32)
      return 0

    lax.fori_loop(0, 40, zrow, 0)

    base = c * EPC + s * SUB
    ne = jnp.where(s < NS - 1, SUB, EPC - (NS - 1) * SUB)
    nk = ne // CH

    zd = [
        pltpu.async_copy(zbuf, agg.at[pl.ds(s * rpw + q * 40, 40)], sem0)
        for q in range(rpw // 40)
    ]
    for d in zd:
      d.wait()

    for g in range(NG):
      goff = pl.multiple_of(g * E + base, 8)
      pltpu.sync_copy(src_hbm.at[pl.ds(goff, SUB)], sbuf)
      pltpu.sync_copy(dst_hbm.at[pl.ds(goff, SUB)], dbuf)
      plsc.subcore_barrier()

      def gather(k, rows, sem):
        pltpu.async_copy(tmp_hbm.at[g].at[sbuf.at[pl.ds(k * CH, CH)]],
                         rows, sem)

      def gwait(rows, sem):
        pltpu.make_async_copy(tmp_hbm.at[g].at[sbuf.at[pl.ds(0, CH)]],
                              rows, sem).wait()

      def dcopy(k, didx):
        for q in range(CH // 16):
          didx[pl.ds(q * 16, 16)] = dbuf[pl.ds(k * CH + q * 16, 16)]

      gather(0, rows0, sem0)

      def pair(j, _):
        a = 2 * j

        @pl.when(a + 1 < nk)
        def _():
          gather(a + 1, rows1, sem1)

        dcopy(a, didx0)
        gwait(rows0, sem0)
        pltpu.sync_copy(rows0, agg.at[didx0], add=True)

        @pl.when(a + 2 < nk)
        def _():
          gather(a + 2, rows0, sem0)

        @pl.when(a + 1 < nk)
        def _():
          dcopy(a + 1, didx1)
          gwait(rows1, sem1)
          pltpu.sync_copy(rows1, agg.at[didx1], add=True)

        return 0

      lax.fori_loop(0, (nk + 1) // 2, pair, 0)
      plsc.subcore_barrier()
      wd = [
          pltpu.async_copy(agg.at[pl.ds(s * rpw + q * CH, CH)],
                           out_hbm.at[c, g, pl.ds(s * rpw + q * CH, CH)],
                           sem1)
          for q in range(rpw // CH)
      ]
      for d in wd:
        d.wait()
      plsc.subcore_barrier()

  return body(tmp_all, src_all, dst_all)


def _tc_pre(degp, gfe0, ofe0, W0):
  grid = (pl.cdiv(N, BN),)

  def body(deg_ref, gfe_ref, ofe_ref, w_ref, tmp_ref, dinv_ref):
    deg = deg_ref[...]
    dinv = lax.rsqrt(deg[0] + deg[1] + 1.0)
    w = w_ref[...]
    for g in range(NG):
      x = gfe_ref[g] if g < C else ofe_ref[...]
      h = jnp.dot(x, w, preferred_element_type=_f32)
      tmp_ref[g, :, :] = h * dinv[g][:, None]
    dinv_ref[...] = dinv

  return pl.pallas_call(
      body,
      grid=grid,
      in_specs=[
          pl.BlockSpec((NC, NG, BN), lambda i: (0, 0, i)),
          pl.BlockSpec((C, BN, D), lambda i: (0, i, 0)),
          pl.BlockSpec((BN, D), lambda i: (i, 0)),
          pl.BlockSpec((D, D), lambda i: (0, 0)),
      ],
      out_specs=[
          pl.BlockSpec((NG, BN, D), lambda i: (0, i, 0)),
          pl.BlockSpec((NG, BN), lambda i: (0, i)),
      ],
      out_shape=[
          jax.ShapeDtypeStruct((NG, N, D), _f32),
          jax.ShapeDtypeStruct((NG, N), _f32),
      ],
  )(degp, gfe0, ofe0, W0)


def _attention_stage(tmp, agg, dinv, label_emb, Wq, bq, b, Wk, bk, Wv, bv,
                     W_next=None):
  final = W_next is None
  grid = (pl.cdiv(N, BN),)
  inv_sqrt_d = 1.0 / math.sqrt(D)

  def body(tmp_ref, agg_ref, dinv_ref, lemb_ref, wq_ref, bq_ref, b_ref,
           wk_ref, bk_ref, wv_ref, bv_ref, *rest):
    if final:
      gfe_ref, ofe_ref = rest
    else:
      (wn_ref, out_ref) = rest
    lq = jnp.dot(lemb_ref[...], wq_ref[...],
                 preferred_element_type=_f32) + bq_ref[...]
    dinv = dinv_ref[...]
    bias = b_ref[...]
    conv = []
    prev = None
    for g in range(NG):
      cum = agg_ref[0, g] + agg_ref[1, g]
      delta = cum if prev is None else cum - prev
      prev = cum
      cg = dinv[g][:, None] * (delta + tmp_ref[g])
      conv.append(cg + bias)
    wk = wk_ref[...]
    wv = wv_ref[...]
    bk = bk_ref[...]
    bv = bv_ref[...]
    scores = []
    vs = []
    for g in range(C):
      kg = jnp.dot(conv[g], wk, preferred_element_type=_f32) + bk
      vs.append(jnp.dot(conv[g], wv, preferred_element_type=_f32) + bv)
      scores.append([
          jnp.sum(kg * lq[a][None, :], axis=1) * inv_sqrt_d for a in range(C)
      ])
    outs = []
    for a in range(C):
      m = scores[0][a]
      for g in range(1, C):
        m = jnp.maximum(m, scores[g][a])
      es = [jnp.exp(scores[g][a] - m) for g in range(C)]
      z = es[0] + es[1] + es[2] + es[3]
      o = (es[0] / z)[:, None] * vs[0]
      for g in range(1, C):
        o = o + (es[g] / z)[:, None] * vs[g]
      outs.append(o)
    if final:
      for a in range(C):
        gfe_ref[a, :, :] = outs[a]
      ofe_ref[...] = conv[C]
    else:
      wn = wn_ref[...]
      for g in range(NG):
        x2 = jnp.maximum(outs[g] if g < C else conv[C], 0.0)
        h2 = jnp.dot(x2, wn, preferred_element_type=_f32)
        out_ref[g, :, :] = h2 * dinv[g][:, None]

  full = lambda shape: pl.BlockSpec(shape, lambda i: tuple(0 for _ in shape))
  in_specs = [
      pl.BlockSpec((NG, BN, D), lambda i: (0, i, 0)),
      pl.BlockSpec((NC, NG, BN, D), lambda i: (0, 0, i, 0)),
      pl.BlockSpec((NG, BN), lambda i: (0, i)),
      full((C, D)), full((D, D)), full((D,)), full((D,)),
      full((D, D)), full((D,)), full((D, D)), full((D,)),
  ]
  args = [tmp, agg, dinv, label_emb, Wq, bq, b, Wk, bk, Wv, bv]
  if final:
    out_specs = [
        pl.BlockSpec((C, BN, D), lambda i: (0, i, 0)),
        pl.BlockSpec((BN, D), lambda i: (i, 0)),
    ]
    out_shape = [
        jax.ShapeDtypeStruct((C, N, D), _f32),
        jax.ShapeDtypeStruct((N, D), _f32),
    ]
  else:
    in_specs.append(full((D, D)))
    args.append(W_next)
    out_specs = [pl.BlockSpec((NG, BN, D), lambda i: (0, i, 0))]
    out_shape = [jax.ShapeDtypeStruct((NG, N, D), _f32)]
  return pl.pallas_call(
      body, grid=grid, in_specs=in_specs, out_specs=out_specs,
      out_shape=out_shape)(*args)


def kernel(graph_feat_emb, ori_feat_emb, label_emb, edge_index_label,
           edge_index_ori, W0, b0, W1, b1, Wq, bq, Wk, bk, Wv, bv):
  eil = edge_index_label.astype(jnp.int32)
  eio = edge_index_ori.astype(jnp.int32)
  src_all = jnp.pad(
      jnp.concatenate([eil[:, 0, :], eio[0:1]], axis=0).reshape(-1), (0, 2048))
  dst_all = jnp.pad(
      jnp.concatenate([eil[:, 1, :], eio[1:2]], axis=0).reshape(-1), (0, 2048))

  degp = _deg_counts(dst_all).reshape(NC, NG, NP)
  tmp1, dinv = _tc_pre(degp, graph_feat_emb, ori_feat_emb, W0)
  agg1 = _edge_aggregate(tmp1, src_all, dst_all)
  (tmp2,) = _attention_stage(tmp1, agg1, dinv, label_emb, Wq, bq, b0,
                             Wk, bk, Wv, bv, W_next=W1)
  agg2 = _edge_aggregate(tmp2, src_all, dst_all)
  gfe_f, ofe_f = _attention_stage(tmp2, agg2, dinv, label_emb, Wq, bq, b1,
                                  Wk, bk, Wv, bv)
  return gfe_f, ofe_f


# speedup vs baseline: 4.6459x; 1.0121x over previous
import functools
import math

import jax
import jax.numpy as jnp
from jax import lax
from jax.experimental import pallas as pl
from jax.experimental.pallas import tpu as pltpu
from jax.experimental.pallas import tpu_sc as plsc

N = 10000
E = 160000
C = 4
D = 128
NG = 5
NP = 10240
BN = 512
NC = 2
NS = 16
CH = 128
EPC = E // NC
SUB = 5120

_f32 = jnp.float32


def _deg_counts(dst_all):
  mesh = plsc.VectorSubcoreMesh(core_axis_name="c", subcore_axis_name="s")

  @functools.partial(
      pl.kernel,
      mesh=mesh,
      out_type=jax.ShapeDtypeStruct((NC * NG * NP,), _f32),
      scratch_types=[
          pltpu.VMEM((SUB,), jnp.int32),
          pltpu.VMEM((CH,), jnp.int32),
          pltpu.VMEM((CH,), _f32),
          pltpu.VMEM((1600,), _f32),
          pltpu.VMEM_SHARED((NG * NP,), _f32),
      ],
  )
  def body(dst_hbm, out_hbm, dbuf, gidx, ones, zb, deg):
    c = lax.axis_index("c")
    s = lax.axis_index("s")
    for q in range(CH // 16):
      ones[pl.ds(q * 16, 16)] = jnp.full((16,), 1.0, _f32)

    def zrow(j, _):
      zb[pl.ds(j * 16, 16)] = jnp.zeros((16,), _f32)
      return 0

    lax.fori_loop(0, 100, zrow, 0)
    for t in range(2):
      pltpu.sync_copy(zb, deg.at[pl.ds(s * 3200 + t * 1600, 1600)])
    plsc.subcore_barrier()

    base = c * EPC + s * SUB
    ne = jnp.where(s < NS - 1, SUB, EPC - (NS - 1) * SUB)
    nk = ne // CH
    for g in range(NG):
      pltpu.sync_copy(
          dst_hbm.at[pl.ds(pl.multiple_of(g * E + base, 8), SUB)], dbuf)

      def chunk(k, _):
        for q in range(CH // 16):
          gidx[pl.ds(q * 16, 16)] = dbuf[pl.ds(k * CH + q * 16, 16)] + g * NP
        pltpu.sync_copy(ones, deg.at[gidx], add=True)
        return 0

      lax.fori_loop(0, nk, chunk, 0)
    plsc.subcore_barrier()
    for t in range(2):
      o = s * 3200 + t * 1600
      pltpu.sync_copy(deg.at[pl.ds(o, 1600)], zb)
      pltpu.sync_copy(zb, out_hbm.at[pl.ds(c * (NG * NP) + o, 1600)])

  return body(dst_all)


def _edge_aggregate(tmp_all, src_all, dst_all):
  mesh = plsc.VectorSubcoreMesh(core_axis_name="c", subcore_axis_name="s")

  @functools.partial(
      pl.kernel,
      mesh=mesh,
      out_type=jax.ShapeDtypeStruct((NC, NG, NP, D), _f32),
      scratch_types=[
          pltpu.VMEM((SUB,), jnp.int32),
          pltpu.VMEM((SUB,), jnp.int32),
          pltpu.VMEM((CH,), jnp.int32),
          pltpu.VMEM((CH,), jnp.int32),
          pltpu.VMEM((CH, D), _f32),
          pltpu.VMEM((CH, D), _f32),
          pltpu.VMEM((40, D), _f32),
          pltpu.VMEM_SHARED((NP, D), _f32),
          pltpu.SemaphoreType.DMA,
          pltpu.SemaphoreType.DMA,
      ],
  )
  def body(tmp_hbm, src_hbm, dst_hbm, out_hbm, sbuf, dbuf, didx0, didx1,
           rows0, rows1, zbuf, agg, sem0, sem1):
    c = lax.axis_index("c")
    s = lax.axis_index("s")
    rpw = NP // NS

    def zrow(r, _):
      for q in range(D // 16):
        zbuf[r, pl.ds(q * 16, 16)] = jnp.zeros((16,), _f32)
      return 0

    lax.fori_loop(0, 40, zrow, 0)

    base = c * EPC + s * SUB
    ne = jnp.where(s < NS - 1, SUB, EPC - (NS - 1) * SUB)
    nk = ne // CH

    zd = [
        pltpu.async_copy(zbuf, agg.at[pl.ds(s * rpw + q * 40, 40)], sem0)
        for q in range(rpw // 40)
    ]
    for d in zd:
      d.wait()
    plsc.subcore_barrier()

    for g in range(NG):
      goff = pl.multiple_of(g * E + base, 8)
      d0 = pltpu.async_copy(src_hbm.at[pl.ds(goff, SUB)], sbuf, sem0)
      d1 = pltpu.async_copy(dst_hbm.at[pl.ds(goff, SUB)], dbuf, sem1)
      d0.wait()
      d1.wait()

      def gather(k, rows, sem):
        pltpu.async_copy(tmp_hbm.at[g].at[sbuf.at[pl.ds(k * CH, CH)]],
                         rows, sem)

      def gwait(rows, sem):
        pltpu.make_async_copy(tmp_hbm.at[g].at[sbuf.at[pl.ds(0, CH)]],
                              rows, sem).wait()

      def dcopy(k, didx):
        for q in range(CH // 16):
          didx[pl.ds(q * 16, 16)] = dbuf[pl.ds(k * CH + q * 16, 16)]

      gather(0, rows0, sem0)

      def pair(j, _):
        a = 2 * j

        @pl.when(a + 1 < nk)
        def _():
          gather(a + 1, rows1, sem1)

        dcopy(a, didx0)
        gwait(rows0, sem0)
        pltpu.sync_copy(rows0, agg.at[didx0], add=True)

        @pl.when(a + 2 < nk)
        def _():
          gather(a + 2, rows0, sem0)

        @pl.when(a + 1 < nk)
        def _():
          dcopy(a + 1, didx1)
          gwait(rows1, sem1)
          pltpu.sync_copy(rows1, agg.at[didx1], add=True)

        return 0

      lax.fori_loop(0, (nk + 1) // 2, pair, 0)
      plsc.subcore_barrier()
      wd = [
          pltpu.async_copy(agg.at[pl.ds(s * rpw + q * CH, CH)],
                           out_hbm.at[c, g, pl.ds(s * rpw + q * CH, CH)],
                           sem1)
          for q in range(rpw // CH)
      ]
      for d in wd:
        d.wait()
      plsc.subcore_barrier()

  return body(tmp_all, src_all, dst_all)


def _tc_pre(degp, gfe0, ofe0, W0):
  grid = (pl.cdiv(N, BN),)

  def body(deg_ref, gfe_ref, ofe_ref, w_ref, tmp_ref, dinv_ref):
    deg = deg_ref[...]
    dinv = lax.rsqrt(deg[0] + deg[1] + 1.0)
    w = w_ref[...]
    for g in range(NG):
      x = gfe_ref[g] if g < C else ofe_ref[...]
      h = jnp.dot(x, w, preferred_element_type=_f32)
      tmp_ref[g, :, :] = h * dinv[g][:, None]
    dinv_ref[...] = dinv

  return pl.pallas_call(
      body,
      grid=grid,
      in_specs=[
          pl.BlockSpec((NC, NG, BN), lambda i: (0, 0, i)),
          pl.BlockSpec((C, BN, D), lambda i: (0, i, 0)),
          pl.BlockSpec((BN, D), lambda i: (i, 0)),
          pl.BlockSpec((D, D), lambda i: (0, 0)),
      ],
      out_specs=[
          pl.BlockSpec((NG, BN, D), lambda i: (0, i, 0)),
          pl.BlockSpec((NG, BN), lambda i: (0, i)),
      ],
      out_shape=[
          jax.ShapeDtypeStruct((NG, N, D), _f32),
          jax.ShapeDtypeStruct((NG, N), _f32),
      ],
  )(degp, gfe0, ofe0, W0)


def _attention_stage(tmp, agg, dinv, label_emb, Wq, bq, b, Wk, bk, Wv, bv,
                     W_next=None):
  final = W_next is None
  grid = (pl.cdiv(N, BN),)
  inv_sqrt_d = 1.0 / math.sqrt(D)

  def body(tmp_ref, agg_ref, dinv_ref, lemb_ref, wq_ref, bq_ref, b_ref,
           wk_ref, bk_ref, wv_ref, bv_ref, *rest):
    if final:
      gfe_ref, ofe_ref = rest
    else:
      (wn_ref, out_ref) = rest
    lq = jnp.dot(lemb_ref[...], wq_ref[...],
                 preferred_element_type=_f32) + bq_ref[...]
    dinv = dinv_ref[...]
    bias = b_ref[...]
    conv = []
    prev = None
    for g in range(NG):
      cum = agg_ref[0, g] + agg_ref[1, g]
      delta = cum if prev is None else cum - prev
      prev = cum
      cg = dinv[g][:, None] * (delta + tmp_ref[g])
      conv.append(cg + bias)
    wk = wk_ref[...]
    wv = wv_ref[...]
    bk = bk_ref[...]
    bv = bv_ref[...]
    scores = []
    vs = []
    for g in range(C):
      kg = jnp.dot(conv[g], wk, preferred_element_type=_f32) + bk
      vs.append(jnp.dot(conv[g], wv, preferred_element_type=_f32) + bv)
      scores.append([
          jnp.sum(kg * lq[a][None, :], axis=1) * inv_sqrt_d for a in range(C)
      ])
    outs = []
    for a in range(C):
      m = scores[0][a]
      for g in range(1, C):
        m = jnp.maximum(m, scores[g][a])
      es = [jnp.exp(scores[g][a] - m) for g in range(C)]
      z = es[0] + es[1] + es[2] + es[3]
      o = (es[0] / z)[:, None] * vs[0]
      for g in range(1, C):
        o = o + (es[g] / z)[:, None] * vs[g]
      outs.append(o)
    if final:
      for a in range(C):
        gfe_ref[a, :, :] = outs[a]
      ofe_ref[...] = conv[C]
    else:
      wn = wn_ref[...]
      for g in range(NG):
        x2 = jnp.maximum(outs[g] if g < C else conv[C], 0.0)
        h2 = jnp.dot(x2, wn, preferred_element_type=_f32)
        out_ref[g, :, :] = h2 * dinv[g][:, None]

  full = lambda shape: pl.BlockSpec(shape, lambda i: tuple(0 for _ in shape))
  in_specs = [
      pl.BlockSpec((NG, BN, D), lambda i: (0, i, 0)),
      pl.BlockSpec((NC, NG, BN, D), lambda i: (0, 0, i, 0)),
      pl.BlockSpec((NG, BN), lambda i: (0, i)),
      full((C, D)), full((D, D)), full((D,)), full((D,)),
      full((D, D)), full((D,)), full((D, D)), full((D,)),
  ]
  args = [tmp, agg, dinv, label_emb, Wq, bq, b, Wk, bk, Wv, bv]
  if final:
    out_specs = [
        pl.BlockSpec((C, BN, D), lambda i: (0, i, 0)),
        pl.BlockSpec((BN, D), lambda i: (i, 0)),
    ]
    out_shape = [
        jax.ShapeDtypeStruct((C, N, D), _f32),
        jax.ShapeDtypeStruct((N, D), _f32),
    ]
  else:
    in_specs.append(full((D, D)))
    args.append(W_next)
    out_specs = [pl.BlockSpec((NG, BN, D), lambda i: (0, i, 0))]
    out_shape = [jax.ShapeDtypeStruct((NG, N, D), _f32)]
  return pl.pallas_call(
      body, grid=grid, in_specs=in_specs, out_specs=out_specs,
      out_shape=out_shape)(*args)


def kernel(graph_feat_emb, ori_feat_emb, label_emb, edge_index_label,
           edge_index_ori, W0, b0, W1, b1, Wq, bq, Wk, bk, Wv, bv):
  eil = edge_index_label.astype(jnp.int32)
  eio = edge_index_ori.astype(jnp.int32)
  src_all = jnp.pad(
      jnp.concatenate([eil[:, 0, :], eio[0:1]], axis=0).reshape(-1), (0, 2048))
  dst_all = jnp.pad(
      jnp.concatenate([eil[:, 1, :], eio[1:2]], axis=0).reshape(-1), (0, 2048))

  degp = _deg_counts(dst_all).reshape(NC, NG, NP)
  tmp1, dinv = _tc_pre(degp, graph_feat_emb, ori_feat_emb, W0)
  agg1 = _edge_aggregate(tmp1, src_all, dst_all)
  (tmp2,) = _attention_stage(tmp1, agg1, dinv, label_emb, Wq, bq, b0,
                             Wk, bk, Wv, bv, W_next=W1)
  agg2 = _edge_aggregate(tmp2, src_all, dst_all)
  gfe_f, ofe_f = _attention_stage(tmp2, agg2, dinv, label_emb, Wq, bq, b1,
                                  Wk, bk, Wv, bv)
  return gfe_f, ofe_f
